# baseline (speedup 1.0000x reference)
import functools

import jax
import jax.numpy as jnp
from jax import lax
from jax.experimental import pallas as pl
from jax.experimental.pallas import tpu as pltpu
from jax.experimental.pallas import tpu_sc as plsc

_NC = 2
_NS = 16


def _blend_kernel(S, BUF, Rb,
                  p_ref, kc_ref, kv_ref, ko_ref, kext_ref):
    j = pl.program_id(1)

    @pl.when(j == 0)
    def _fill():
        kext_ref[pl.ds(Rb, S), :] = kv_ref[0, 0, :, :]

    p = p_ref[0]
    base = (j * Rb - p) % BUF
    v0 = (base + Rb) % BUF - Rb
    v0 = jnp.clip(v0, -Rb, S)

    rows = lax.broadcasted_iota(jnp.int32, (Rb, 1), 0)
    idx = base + rows
    idx = jnp.where(idx >= BUF, idx - BUF, idx)
    mask = idx < S

    slab = kext_ref[pl.ds(Rb + v0, Rb), :]
    ko_ref[0, 0, :, :] = jnp.where(mask, slab, kc_ref[0, 0, :, :])


def _sc_v_body(S, BUF, D, C,
               vc_ref, vv_ref, p_ref, vo_ref,
               rows_ref, idx_ref, pvmem_ref, sem):
    wid = lax.axis_index("s") * _NC + lax.axis_index("c")

    pltpu.sync_copy(p_ref, pvmem_ref)
    p = pvmem_ref[...][0]

    nch = BUF // C
    first = pltpu.make_async_copy(
        vc_ref.at[0, wid, pl.ds(0, C)], rows_ref.at[0], sem.at[0])
    first.start()

    def copy_chunk(c, _):
        cur = lax.rem(c, 2)
        nxt = lax.rem(c + 1, 2)

        @pl.when(c + 1 < nch)
        def _():
            pltpu.make_async_copy(
                vc_ref.at[0, wid, pl.ds((c + 1) * C, C)],
                rows_ref.at[nxt], sem.at[nxt]).start()

        pltpu.make_async_copy(
            vc_ref.at[0, wid, pl.ds(c * C, C)], rows_ref.at[cur],
            sem.at[cur]).wait()
        pltpu.sync_copy(rows_ref.at[cur], vo_ref.at[0, wid, pl.ds(c * C, C)])
        return 0

    lax.fori_loop(0, nch, copy_chunk, 0)

    def build_idx(i, base):
        vec = base + lax.iota(jnp.int32, 16) + i * 16
        vec = jnp.where(vec >= BUF, vec - BUF, vec)
        idx_ref[pl.ds(i * 16, 16)] = vec
        return base

    for c in range(S // C):
        lax.fori_loop(0, C // 16, build_idx, p + c * C)
        pltpu.sync_copy(vv_ref.at[0, wid, pl.ds(c * C, C)], rows_ref.at[0])
        pltpu.async_copy(rows_ref.at[0], vo_ref.at[0, wid].at[idx_ref],
                         sem.at[0]).wait()


@jax.jit
def kernel(k_cache, v_cache, k_val, v_val, input_pos):
    B, H, BUF, D = k_cache.shape
    S = k_val.shape[2]
    Rb = 8192
    C = 256

    p = jnp.asarray(input_pos, jnp.int32).reshape((1,)) % BUF

    mesh = plsc.VectorSubcoreMesh(core_axis_name="c", subcore_axis_name="s")
    v_new = pl.kernel(
        functools.partial(_sc_v_body, S, BUF, D, C),
        out_type=jax.ShapeDtypeStruct(v_cache.shape, v_cache.dtype),
        mesh=mesh,
        scratch_types=[
            pltpu.VMEM((2, C, D), jnp.float32),
            pltpu.VMEM((C,), jnp.int32),
            pltpu.VMEM((16,), jnp.int32),
            pltpu.SemaphoreType.DMA((2,)),
        ],
    )(v_cache, v_val, jnp.broadcast_to(p, (16,)))

    cache_spec = pl.BlockSpec((1, 1, Rb, D), lambda h, j, p_ref: (0, h, j, 0))
    val_spec = pl.BlockSpec((1, 1, S, D), lambda h, j, p_ref: (0, h, 0, 0))
    k_new = pl.pallas_call(
        functools.partial(_blend_kernel, S, BUF, Rb),
        grid_spec=pltpu.PrefetchScalarGridSpec(
            num_scalar_prefetch=1,
            grid=(H, BUF // Rb),
            in_specs=[cache_spec, val_spec],
            out_specs=cache_spec,
            scratch_shapes=[pltpu.VMEM((S + 2 * Rb, D), jnp.float32)],
        ),
        out_shape=jax.ShapeDtypeStruct(k_cache.shape, k_cache.dtype),
        compiler_params=pltpu.CompilerParams(
            dimension_semantics=("arbitrary", "arbitrary"),
        ),
    )(p, k_cache, k_val)

    return (k_new, v_new)


# speedup vs baseline: 1.0197x; 1.0197x over previous
import functools

import jax
import jax.numpy as jnp
from jax import lax
from jax.experimental import pallas as pl
from jax.experimental.pallas import tpu as pltpu
from jax.experimental.pallas import tpu_sc as plsc

_NC = 2
_NS = 16


def _blend_kernel(S, BUF, Rb,
                  p_ref, kc_ref, kv_ref, ko_ref, kext_ref):
    j = pl.program_id(1)

    @pl.when(j == 0)
    def _fill():
        kext_ref[pl.ds(Rb, S), :] = kv_ref[0, 0, :, :]

    p = p_ref[0]
    base = (j * Rb - p) % BUF
    v0 = (base + Rb) % BUF - Rb
    v0 = jnp.clip(v0, -Rb, S)

    rows = lax.broadcasted_iota(jnp.int32, (Rb, 1), 0)
    idx = base + rows
    idx = jnp.where(idx >= BUF, idx - BUF, idx)
    mask = idx < S

    slab = kext_ref[pl.ds(Rb + v0, Rb), :]
    ko_ref[0, 0, :, :] = jnp.where(mask, slab, kc_ref[0, 0, :, :])


def _copy_top_kernel(vc_ref, vo_ref):
    vo_ref[...] = vc_ref[...]


def _sc_v_body(S, BUF, BUFSC, C,
               vo_ref, vc_ref, vv_ref, p_ref,
               rows_ref, idx_ref, pvmem_ref, sem):
    wid = lax.axis_index("s") * _NC + lax.axis_index("c")

    pltpu.sync_copy(p_ref, pvmem_ref)
    p = pvmem_ref[...][0]

    nch = BUFSC // C
    pltpu.make_async_copy(
        vc_ref.at[0, wid, pl.ds(0, C)], rows_ref.at[0], sem.at[0]).start()

    def copy_chunk(c, _):
        cur = lax.rem(c, 2)
        nxt = lax.rem(c + 1, 2)

        @pl.when(c + 1 < nch)
        def _():
            pltpu.make_async_copy(
                vc_ref.at[0, wid, pl.ds((c + 1) * C, C)],
                rows_ref.at[nxt], sem.at[nxt]).start()

        pltpu.make_async_copy(
            vc_ref.at[0, wid, pl.ds(c * C, C)], rows_ref.at[cur],
            sem.at[cur]).wait()
        pltpu.sync_copy(rows_ref.at[cur], vo_ref.at[0, wid, pl.ds(c * C, C)])
        return 0

    lax.fori_loop(0, nch, copy_chunk, 0)

    def build_idx(i, base):
        vec = base + lax.iota(jnp.int32, 16) + i * 16
        vec = jnp.where(vec >= BUF, vec - BUF, vec)
        idx_ref[pl.ds(i * 16, 16)] = vec
        return base

    for c in range(S // C):
        lax.fori_loop(0, C // 16, build_idx, p + c * C)
        pltpu.sync_copy(vv_ref.at[0, wid, pl.ds(c * C, C)], rows_ref.at[0])
        pltpu.async_copy(rows_ref.at[0], vo_ref.at[0, wid].at[idx_ref],
                         sem.at[0]).wait()


@jax.jit
def kernel(k_cache, v_cache, k_val, v_val, input_pos):
    B, H, BUF, D = k_cache.shape
    S = k_val.shape[2]
    Rb = 8192
    Rt = 5120
    BUFSC = 6144
    C = 256

    p = jnp.asarray(input_pos, jnp.int32).reshape((1,)) % BUF

    top_spec = pl.BlockSpec(
        (1, 1, Rt, D), lambda h, j: (0, h, BUFSC // Rt + j, 0))
    vp = pl.pallas_call(
        _copy_top_kernel,
        grid=(H, (BUF - BUFSC) // Rt),
        in_specs=[top_spec],
        out_specs=top_spec,
        out_shape=jax.ShapeDtypeStruct(v_cache.shape, v_cache.dtype),
        compiler_params=pltpu.CompilerParams(
            dimension_semantics=("arbitrary", "arbitrary"),
        ),
    )(v_cache)

    rv = jax.new_ref(vp)
    mesh = plsc.VectorSubcoreMesh(core_axis_name="c", subcore_axis_name="s")
    pl.kernel(
        functools.partial(_sc_v_body, S, BUF, BUFSC, C),
        mesh=mesh,
        scratch_types=[
            pltpu.VMEM((2, C, D), jnp.float32),
            pltpu.VMEM((C,), jnp.int32),
            pltpu.VMEM((16,), jnp.int32),
            pltpu.SemaphoreType.DMA((2,)),
        ],
    )(rv, v_cache, v_val, jnp.broadcast_to(p, (16,)))

    cache_spec = pl.BlockSpec((1, 1, Rb, D), lambda h, j, p_ref: (0, h, j, 0))
    val_spec = pl.BlockSpec((1, 1, S, D), lambda h, j, p_ref: (0, h, 0, 0))
    k_new = pl.pallas_call(
        functools.partial(_blend_kernel, S, BUF, Rb),
        grid_spec=pltpu.PrefetchScalarGridSpec(
            num_scalar_prefetch=1,
            grid=(H, BUF // Rb),
            in_specs=[cache_spec, val_spec],
            out_specs=cache_spec,
            scratch_shapes=[pltpu.VMEM((S + 2 * Rb, D), jnp.float32)],
        ),
        out_shape=jax.ShapeDtypeStruct(k_cache.shape, k_cache.dtype),
        compiler_params=pltpu.CompilerParams(
            dimension_semantics=("arbitrary", "arbitrary"),
        ),
    )(p, k_cache, k_val)

    return (k_new, jax.freeze(rv))


# speedup vs baseline: 1.0271x; 1.0073x over previous
import functools

import jax
import jax.numpy as jnp
from jax import lax
from jax.experimental import pallas as pl
from jax.experimental.pallas import tpu as pltpu
from jax.experimental.pallas import tpu_sc as plsc

_NC = 2
_NS = 16


def _blend_kernel(S, BUF, Rb,
                  p_ref, kc_ref, kv_ref, ko_ref, kext_ref):
    j = pl.program_id(1)

    @pl.when(j == 0)
    def _fill():
        kext_ref[pl.ds(Rb, S), :] = kv_ref[0, 0, :, :]

    p = p_ref[0]
    base = (j * Rb - p) % BUF
    v0 = (base + Rb) % BUF - Rb
    v0 = jnp.clip(v0, -Rb, S)

    rows = lax.broadcasted_iota(jnp.int32, (Rb, 1), 0)
    idx = base + rows
    idx = jnp.where(idx >= BUF, idx - BUF, idx)
    mask = idx < S

    slab = kext_ref[pl.ds(Rb + v0, Rb), :]
    ko_ref[0, 0, :, :] = jnp.where(mask, slab, kc_ref[0, 0, :, :])


def _copy_top_kernel(vc_ref, vo_ref):
    vo_ref[...] = vc_ref[...]


def _sc_v_body(S, BUF, BUFSC, C,
               vo_ref, vc_ref, vv_ref, p_ref,
               rows_ref, idx_ref, pvmem_ref, sem):
    wid = lax.axis_index("s") * _NC + lax.axis_index("c")

    pltpu.sync_copy(p_ref, pvmem_ref)
    p = pvmem_ref[...][0]

    nch = BUFSC // C
    pltpu.make_async_copy(
        vc_ref.at[0, wid, pl.ds(0, C)], rows_ref.at[0], sem.at[0]).start()

    def copy_chunk(c, _):
        cur = lax.rem(c, 2)
        nxt = lax.rem(c + 1, 2)

        @pl.when(c + 1 < nch)
        def _():
            pltpu.make_async_copy(
                vc_ref.at[0, wid, pl.ds((c + 1) * C, C)],
                rows_ref.at[nxt], sem.at[nxt]).start()

        pltpu.make_async_copy(
            vc_ref.at[0, wid, pl.ds(c * C, C)], rows_ref.at[cur],
            sem.at[cur]).wait()
        pltpu.sync_copy(rows_ref.at[cur], vo_ref.at[0, wid, pl.ds(c * C, C)])
        return 0

    lax.fori_loop(0, nch, copy_chunk, 0)

    def build_idx(i, base):
        vec = base + lax.iota(jnp.int32, 16) + i * 16
        vec = jnp.where(vec >= BUF, vec - BUF, vec)
        idx_ref[pl.ds(i * 16, 16)] = vec
        return base

    for c in range(S // C):
        lax.fori_loop(0, C // 16, build_idx, p + c * C)
        pltpu.sync_copy(vv_ref.at[0, wid, pl.ds(c * C, C)], rows_ref.at[0])
        pltpu.async_copy(rows_ref.at[0], vo_ref.at[0, wid].at[idx_ref],
                         sem.at[0]).wait()


@jax.jit
def kernel(k_cache, v_cache, k_val, v_val, input_pos):
    B, H, BUF, D = k_cache.shape
    S = k_val.shape[2]
    Rb = 8192
    Rt = 6144
    BUFSC = 4096
    C = 256

    p = jnp.asarray(input_pos, jnp.int32).reshape((1,)) % BUF

    top_spec = pl.BlockSpec(
        (1, 1, Rt, D), lambda h, j: (0, h, BUFSC // Rt + j, 0))
    vp = pl.pallas_call(
        _copy_top_kernel,
        grid=(H, (BUF - BUFSC) // Rt),
        in_specs=[top_spec],
        out_specs=top_spec,
        out_shape=jax.ShapeDtypeStruct(v_cache.shape, v_cache.dtype),
        compiler_params=pltpu.CompilerParams(
            dimension_semantics=("arbitrary", "arbitrary"),
        ),
    )(v_cache)

    rv = jax.new_ref(vp)
    mesh = plsc.VectorSubcoreMesh(core_axis_name="c", subcore_axis_name="s")
    pl.kernel(
        functools.partial(_sc_v_body, S, BUF, BUFSC, C),
        mesh=mesh,
        scratch_types=[
            pltpu.VMEM((2, C, D), jnp.float32),
            pltpu.VMEM((C,), jnp.int32),
            pltpu.VMEM((16,), jnp.int32),
            pltpu.SemaphoreType.DMA((2,)),
        ],
    )(rv, v_cache, v_val, jnp.broadcast_to(p, (16,)))

    cache_spec = pl.BlockSpec((1, 1, Rb, D), lambda h, j, p_ref: (0, h, j, 0))
    val_spec = pl.BlockSpec((1, 1, S, D), lambda h, j, p_ref: (0, h, 0, 0))
    k_new = pl.pallas_call(
        functools.partial(_blend_kernel, S, BUF, Rb),
        grid_spec=pltpu.PrefetchScalarGridSpec(
            num_scalar_prefetch=1,
            grid=(H, BUF // Rb),
            in_specs=[cache_spec, val_spec],
            out_specs=cache_spec,
            scratch_shapes=[pltpu.VMEM((S + 2 * Rb, D), jnp.float32)],
        ),
        out_shape=jax.ShapeDtypeStruct(k_cache.shape, k_cache.dtype),
        compiler_params=pltpu.CompilerParams(
            dimension_semantics=("arbitrary", "arbitrary"),
        ),
    )(p, k_cache, k_val)

    return (k_new, jax.freeze(rv))
